# baseline (speedup 1.0000x reference)
import functools

import jax
import jax.numpy as jnp
from jax import lax
from jax.experimental import pallas as pl
from jax.experimental.pallas import tpu as pltpu
from jax.experimental.pallas import tpu_sc as plsc

VOCAB = 1000000
EMB = 64
NUM_CORES = 2
NUM_SUBCORES = 16
NW = NUM_CORES * NUM_SUBCORES
CHUNK = 128


def _make_gather(total):
    per_w = total // NW
    nchunk = per_w // CHUNK

    mesh = plsc.VectorSubcoreMesh(core_axis_name="c", subcore_axis_name="s")

    @functools.partial(
        pl.kernel,
        mesh=mesh,
        compiler_params=pltpu.CompilerParams(use_tc_tiling_on_sc=False),
        out_type=jax.ShapeDtypeStruct((total, EMB), jnp.float32),
        scratch_types=[
            pltpu.VMEM((nchunk, CHUNK), jnp.int32),
            pltpu.VMEM((CHUNK, EMB), jnp.float32),
            pltpu.SemaphoreType.DMA,
        ],
    )
    def gather_kernel(idx_hbm, table_hbm, out_hbm, idx_v, rows_v, sem):
        wid = lax.axis_index("s") * NUM_CORES + lax.axis_index("c")
        base = wid * per_w
        pltpu.sync_copy(idx_hbm.at[wid], idx_v)

        def body(j, _):
            pltpu.async_copy(table_hbm.at[idx_v.at[j]], rows_v, sem).wait()
            pltpu.sync_copy(rows_v, out_hbm.at[pl.ds(base + j * CHUNK, CHUNK)])
            return ()

        lax.fori_loop(0, nchunk, body, (), unroll=False)

    return gather_kernel


def kernel(inputs, shared_weights):
    b, s = inputs.shape
    total = b * s
    idx = inputs.reshape(NW, total // (NW * CHUNK), CHUNK)
    out = _make_gather(total)(idx, shared_weights)
    return out.reshape(b, s, EMB)


# speedup vs baseline: 1.1174x; 1.1174x over previous
import functools

import jax
import jax.numpy as jnp
from jax import lax
from jax.experimental import pallas as pl
from jax.experimental.pallas import tpu as pltpu
from jax.experimental.pallas import tpu_sc as plsc

VOCAB = 1000000
EMB = 64
NUM_CORES = 2
NUM_SUBCORES = 16
NW = NUM_CORES * NUM_SUBCORES
CHUNK = 128


def _make_gather(total):
    per_w = total // NW
    nchunk = per_w // CHUNK

    mesh = plsc.VectorSubcoreMesh(core_axis_name="c", subcore_axis_name="s")

    nbuf = 8
    assert (nchunk - nbuf) % nbuf == 0

    @functools.partial(
        pl.kernel,
        mesh=mesh,
        compiler_params=pltpu.CompilerParams(use_tc_tiling_on_sc=False),
        out_type=jax.ShapeDtypeStruct((total, EMB), jnp.float32),
        scratch_types=[
            pltpu.VMEM((nchunk, CHUNK), jnp.int32),
            pltpu.VMEM((nbuf, CHUNK, EMB), jnp.float32),
            pltpu.SemaphoreType.DMA((nbuf,)),
        ],
    )
    def gather_kernel(idx_hbm, table_hbm, out_hbm, idx_v, rows_v, gsem):
        wid = lax.axis_index("s") * NUM_CORES + lax.axis_index("c")
        base = wid * per_w
        pltpu.sync_copy(idx_hbm.at[wid], idx_v)

        def gather(j, b):
            return pltpu.make_async_copy(
                table_hbm.at[idx_v.at[j]], rows_v.at[b], gsem.at[b])

        def put(j, b):
            pltpu.sync_copy(
                rows_v.at[b], out_hbm.at[pl.ds(base + j * CHUNK, CHUNK)])

        for b in range(nbuf):
            gather(b, b).start()

        def body(g, _):
            for b in range(nbuf):
                j = g * nbuf + b
                gather(j, b).wait()
                put(j, b)
                gather(j + nbuf, b).start()
            return ()

        lax.fori_loop(0, (nchunk - nbuf) // nbuf, body, (), unroll=False)

        for b in range(nbuf):
            j = nchunk - nbuf + b
            gather(j, b).wait()
            put(j, b)

    return gather_kernel


def kernel(inputs, shared_weights):
    b, s = inputs.shape
    total = b * s
    idx = inputs.reshape(NW, total // (NW * CHUNK), CHUNK)
    out = _make_gather(total)(idx, shared_weights)
    return out.reshape(b, s, EMB)
